# baseline (speedup 1.0000x reference)
import functools

import jax
import jax.numpy as jnp
from jax import lax
from jax.experimental import pallas as pl
from jax.experimental.pallas import tpu as pltpu
from jax.experimental.pallas import tpu_sc as plsc

N = 10000
E = 160000
DE = 16
H = 256

NC, NS = 2, 16
NW = NC * NS
CH = 128
EP = 163840
EPW = EP // NW
GCH = EPW // CH

BPT = 320
ACC_R = BPT + 8
SCH2 = 64
NPAD = NW * BPT

BN = 2000
BE = 1280


@functools.cache
def _sc_kernels():
    mesh = plsc.VectorSubcoreMesh(core_axis_name="c", subcore_axis_name="s")

    @functools.partial(
        pl.kernel,
        mesh=mesh,
        out_type=jax.ShapeDtypeStruct((EP, H), jnp.float32),
        scratch_types=[
            pltpu.VMEM((CH,), jnp.int32),
            pltpu.VMEM((CH, H), jnp.float32),
            pltpu.SemaphoreType.DMA,
        ],
    )
    def gather(table_hbm, idx_hbm, out_hbm, idx_v, rows_v, sem):
        wid = lax.axis_index("s") * NC + lax.axis_index("c")
        w0 = wid * EPW

        def body(i, carry):
            base = w0 + i * CH
            pltpu.sync_copy(idx_hbm.at[pl.ds(base, CH)], idx_v)
            pltpu.async_copy(table_hbm.at[idx_v], rows_v, sem).wait()
            pltpu.sync_copy(rows_v, out_hbm.at[pl.ds(base, CH)])
            return carry

        lax.fori_loop(0, GCH, body, 0)

    @functools.partial(
        pl.kernel,
        mesh=mesh,
        out_type=jax.ShapeDtypeStruct((NPAD, H), jnp.float32),
        scratch_types=[
            pltpu.VMEM((56,), jnp.int32),
            pltpu.VMEM((SCH2 + 16,), jnp.int32),
            pltpu.VMEM((SCH2, H), jnp.float32),
            pltpu.VMEM((ACC_R, H), jnp.float32),
        ],
    )
    def scatter_add(vals_hbm, idx_hbm, binb_hbm, out_hbm, bv, iv, rows_v, acc):
        t = lax.axis_index("s") * NC + lax.axis_index("c")
        base_node = t * BPT
        zv = jnp.zeros((16,), jnp.float32)

        def zr(r, carry):
            for k in range(H // 16):
                acc[r, pl.ds(k * 16, 16)] = zv
            return carry

        lax.fori_loop(0, ACC_R, zr, 0)

        pltpu.sync_copy(binb_hbm.at[pl.ds(0, 40)], bv.at[pl.ds(0, 40)])
        lo = bv[pl.ds(t, 16)][0]
        hi = bv[pl.ds(t + 1, 16)][0]
        lo_al = (lo // 8) * 8
        nch = (hi - lo_al + SCH2 - 1) // SCH2

        def body(i, carry):
            base = lo_al + i * SCH2
            pltpu.sync_copy(idx_hbm.at[pl.ds(base, SCH2)],
                            iv.at[pl.ds(0, SCH2)])
            pltpu.sync_copy(vals_hbm.at[pl.ds(base, SCH2)], rows_v)

            def inner(e, c2):
                r = iv[pl.ds(e, 16)][0] - base_node
                r = jnp.where((r < 0) | (r >= BPT), BPT, r)
                for k in range(H // 16):
                    plsc.addupdate(acc.at[r, pl.ds(k * 16, 16)],
                                   rows_v[e, pl.ds(k * 16, 16)])
                return c2

            lax.fori_loop(0, SCH2, inner, 0)
            return carry

        lax.fori_loop(0, nch, body, 0)
        pltpu.sync_copy(acc.at[pl.ds(0, BPT)],
                        out_hbm.at[pl.ds(base_node, BPT)])

    return gather, scatter_add


def _mm_body(x_ref, w_ref, b_ref, o_ref):
    o_ref[...] = (jnp.dot(x_ref[...], w_ref[...],
                          preferred_element_type=jnp.float32) + b_ref[...])


def _mm(x, w, b):
    n, k = x.shape
    m = w.shape[1]
    return pl.pallas_call(
        _mm_body,
        grid=(n // BN,),
        in_specs=[pl.BlockSpec((BN, k), lambda i: (i, 0)),
                  pl.BlockSpec((k, m), lambda i: (0, 0)),
                  pl.BlockSpec((1, m), lambda i: (0, 0))],
        out_specs=pl.BlockSpec((BN, m), lambda i: (i, 0)),
        out_shape=jax.ShapeDtypeStruct((n, m), jnp.float32),
    )(x, w, b.reshape(1, m))


def _mm2_body(x_ref, wa_ref, wb_ref, oa_ref, ob_ref):
    x = x_ref[...]
    oa_ref[...] = jnp.dot(x, wa_ref[...], preferred_element_type=jnp.float32)
    ob_ref[...] = jnp.dot(x, wb_ref[...], preferred_element_type=jnp.float32)


def _mm2(x, wa, wb):
    n, k = x.shape
    m = wa.shape[1]
    return pl.pallas_call(
        _mm2_body,
        grid=(n // BN,),
        in_specs=[pl.BlockSpec((BN, k), lambda i: (i, 0)),
                  pl.BlockSpec((k, m), lambda i: (0, 0)),
                  pl.BlockSpec((k, m), lambda i: (0, 0))],
        out_specs=[pl.BlockSpec((BN, m), lambda i: (i, 0)),
                   pl.BlockSpec((BN, m), lambda i: (i, 0))],
        out_shape=[jax.ShapeDtypeStruct((n, m), jnp.float32),
                   jax.ShapeDtypeStruct((n, m), jnp.float32)],
    )(x, wa, wb)


def _edge_body(xd_ref, xs_ref, ea_ref, wee_ref, w1c_ref, bee_ref, b1_ref,
               w2_ref, b2_ref, o_ref):
    wf = jnp.dot(wee_ref[...], w1c_ref[...], preferred_element_type=jnp.float32)
    bf = (jnp.dot(bee_ref[...], w1c_ref[...],
                  preferred_element_type=jnp.float32) + b1_ref[...])
    m1 = (xd_ref[...] + xs_ref[...]
          + jnp.dot(ea_ref[...], wf, preferred_element_type=jnp.float32) + bf)
    m1 = jnp.maximum(m1, 0.0)
    m2 = jnp.dot(m1, w2_ref[...], preferred_element_type=jnp.float32) + b2_ref[...]
    o_ref[...] = jnp.maximum(m2, 0.0)


def _edge_mlp(xd, xs, ea, W_ee, W1c, b_ee, b1, W2, b2):
    return pl.pallas_call(
        _edge_body,
        grid=(EP // BE,),
        in_specs=[pl.BlockSpec((BE, H), lambda i: (i, 0)),
                  pl.BlockSpec((BE, H), lambda i: (i, 0)),
                  pl.BlockSpec((BE, DE), lambda i: (i, 0)),
                  pl.BlockSpec((DE, H), lambda i: (0, 0)),
                  pl.BlockSpec((H, H), lambda i: (0, 0)),
                  pl.BlockSpec((1, H), lambda i: (0, 0)),
                  pl.BlockSpec((1, H), lambda i: (0, 0)),
                  pl.BlockSpec((H, H), lambda i: (0, 0)),
                  pl.BlockSpec((1, H), lambda i: (0, 0))],
        out_specs=pl.BlockSpec((BE, H), lambda i: (i, 0)),
        out_shape=jax.ShapeDtypeStruct((EP, H), jnp.float32),
    )(xd, xs, ea, W_ee, W1c, b_ee.reshape(1, H), b1.reshape(1, H),
      W2, b2.reshape(1, H))


def _upd_body(h_ref, a_ref, wa_ref, wb_ref, b_ref, o_ref):
    t = (jnp.dot(h_ref[...], wa_ref[...], preferred_element_type=jnp.float32)
         + jnp.dot(a_ref[...], wb_ref[...], preferred_element_type=jnp.float32)
         + b_ref[...])
    o_ref[...] = jnp.maximum(t, 0.0)


def _update(h, aggr, wa, wb, b):
    return pl.pallas_call(
        _upd_body,
        grid=(N // BN,),
        in_specs=[pl.BlockSpec((BN, H), lambda i: (i, 0)),
                  pl.BlockSpec((BN, H), lambda i: (i, 0)),
                  pl.BlockSpec((H, H), lambda i: (0, 0)),
                  pl.BlockSpec((H, H), lambda i: (0, 0)),
                  pl.BlockSpec((1, H), lambda i: (0, 0))],
        out_specs=pl.BlockSpec((BN, H), lambda i: (i, 0)),
        out_shape=jax.ShapeDtypeStruct((N, H), jnp.float32),
    )(h, aggr, wa, wb, b.reshape(1, H))


def _ro_body(h_ref, wr1_ref, br1_ref, wr2_ref, br2_ref, o_ref):
    g = jnp.sum(h_ref[...], axis=0, keepdims=True) * (1.0 / N)
    t = (jnp.dot(g, wr1_ref[...], preferred_element_type=jnp.float32)
         + br1_ref[...])
    t = jnp.maximum(t, 0.0)
    o_ref[...] = (jnp.dot(t, wr2_ref[...], preferred_element_type=jnp.float32)
                  + br2_ref[...])


def _readout(h, Wr1, br1, Wr2, br2):
    R = Wr1.shape[1]
    OUT = Wr2.shape[1]
    return pl.pallas_call(
        _ro_body,
        grid=(1,),
        in_specs=[pl.BlockSpec((N, H), lambda i: (0, 0)),
                  pl.BlockSpec((H, R), lambda i: (0, 0)),
                  pl.BlockSpec((1, R), lambda i: (0, 0)),
                  pl.BlockSpec((R, OUT), lambda i: (0, 0)),
                  pl.BlockSpec((1, OUT), lambda i: (0, 0))],
        out_specs=pl.BlockSpec((1, OUT), lambda i: (0, 0)),
        out_shape=jax.ShapeDtypeStruct((1, OUT), jnp.float32),
    )(h, Wr1, br1.reshape(1, R), Wr2, br2.reshape(1, OUT))


def kernel(x, edge_index, edge_attr, batch, W_ne, b_ne, W_ee, b_ee,
           W1_0, b1_0, W2_0, b2_0, Wu_0, bu_0,
           W1_1, b1_1, W2_1, b2_1, Wu_1, bu_1,
           W1_2, b1_2, W2_2, b2_2, Wu_2, bu_2,
           Wr1, br1, Wr2, br2):
    src = edge_index[0]
    dst = edge_index[1]
    perm = jnp.argsort(dst).astype(jnp.int32)
    dst_srt = jnp.take(dst, perm)
    src_srt = jnp.take(src, perm)
    binb = jnp.searchsorted(
        dst_srt, jnp.arange(33, dtype=jnp.int32) * BPT).astype(jnp.int32)
    binb = jnp.concatenate([binb, jnp.full((7,), E, jnp.int32)])

    pad = EP - E
    zpad = jnp.zeros((pad,), jnp.int32)
    dst_g = jnp.concatenate([dst_srt, zpad])
    src_g = jnp.concatenate([src_srt, zpad])
    dst_sc = jnp.concatenate([dst_srt, jnp.full((pad,), 1 << 30, jnp.int32)])
    ea_srt = jnp.concatenate(
        [jnp.take(edge_attr, perm, axis=0), jnp.zeros((pad, DE), jnp.float32)])

    _gather_nodes, _sc_scatter_add = _sc_kernels()
    h = _mm(x, W_ne, b_ne)
    layers = [(W1_0, b1_0, W2_0, b2_0, Wu_0, bu_0),
              (W1_1, b1_1, W2_1, b2_1, Wu_1, bu_1),
              (W1_2, b1_2, W2_2, b2_2, Wu_2, bu_2)]
    for (W1, b1, W2, b2, Wu, bu) in layers:
        Hd, Hs = _mm2(h, W1[:H], W1[H:2 * H])
        Xd = _gather_nodes(Hd, dst_g)
        Xs = _gather_nodes(Hs, src_g)
        m2 = _edge_mlp(Xd, Xs, ea_srt, W_ee, W1[2 * H:], b_ee, b1, W2, b2)
        aggr = _sc_scatter_add(m2, dst_sc, binb)
        h = _update(h, aggr[:N], Wu[:H], Wu[H:], bu)
    return _readout(h, Wr1, br1, Wr2, br2)


# speedup vs baseline: 1.1789x; 1.1789x over previous
import functools

import jax
import jax.numpy as jnp
from jax import lax
from jax.experimental import pallas as pl
from jax.experimental.pallas import tpu as pltpu
from jax.experimental.pallas import tpu_sc as plsc

N = 10000
E = 160000
DE = 16
H = 256

NC, NS = 2, 16
NW = NC * NS
CH = 128
EP = 163840
EPW = EP // NW
GCH = EPW // CH

BPT = 320
ACC_R = BPT + 8
SCH2 = 64
NPAD = NW * BPT

BN = 2000
BE = 1280


@functools.cache
def _sc_kernels():
    mesh = plsc.VectorSubcoreMesh(core_axis_name="c", subcore_axis_name="s")

    @functools.partial(
        pl.kernel,
        mesh=mesh,
        out_type=jax.ShapeDtypeStruct((EP, H), jnp.float32),
        scratch_types=[
            pltpu.VMEM((CH,), jnp.int32),
            pltpu.VMEM((CH,), jnp.int32),
            pltpu.VMEM((CH, H), jnp.float32),
            pltpu.VMEM((CH, H), jnp.float32),
            pltpu.SemaphoreType.DMA,
            pltpu.SemaphoreType.DMA,
            pltpu.SemaphoreType.DMA,
            pltpu.SemaphoreType.DMA,
        ],
    )
    def gather(table_hbm, idx_hbm, out_hbm, idx0, idx1, rows0, rows1,
               sg0, sg1, so0, so1):
        wid = lax.axis_index("s") * NC + lax.axis_index("c")
        w0 = wid * EPW
        idxb = [idx0, idx1]
        rowsb = [rows0, rows1]
        sg = [sg0, sg1]
        so = [so0, so1]
        gh = [None, None]
        oh = [None, None]
        for c in range(GCH):
            b = c & 1
            if oh[b] is not None:
                oh[b].wait()
                oh[b] = None
            base = w0 + c * CH
            pltpu.sync_copy(idx_hbm.at[pl.ds(base, CH)], idxb[b])
            gh[b] = pltpu.async_copy(table_hbm.at[idxb[b]], rowsb[b], sg[b])
            ob = 1 - b
            if gh[ob] is not None:
                gh[ob].wait()
                gh[ob] = None
                pbase = w0 + (c - 1) * CH
                oh[ob] = pltpu.async_copy(rowsb[ob],
                                          out_hbm.at[pl.ds(pbase, CH)], so[ob])
        bl = (GCH - 1) & 1
        gh[bl].wait()
        pltpu.sync_copy(rowsb[bl], out_hbm.at[pl.ds(w0 + (GCH - 1) * CH, CH)])
        if oh[1 - bl] is not None:
            oh[1 - bl].wait()

    @functools.partial(
        pl.kernel,
        mesh=mesh,
        out_type=jax.ShapeDtypeStruct((NPAD, H), jnp.float32),
        scratch_types=[
            pltpu.VMEM((56,), jnp.int32),
            pltpu.VMEM((SCH2 + 16,), jnp.int32),
            pltpu.VMEM((SCH2 + 16,), jnp.int32),
            pltpu.VMEM((SCH2, H), jnp.float32),
            pltpu.VMEM((SCH2, H), jnp.float32),
            pltpu.VMEM((ACC_R, H), jnp.float32),
            pltpu.SemaphoreType.DMA,
            pltpu.SemaphoreType.DMA,
            pltpu.SemaphoreType.DMA,
            pltpu.SemaphoreType.DMA,
        ],
    )
    def scatter_add(vals_hbm, idx_hbm, binb_hbm, out_hbm, bv, iv0, iv1,
                    rows0, rows1, acc, si0, si1, sv0, sv1):
        t = lax.axis_index("s") * NC + lax.axis_index("c")
        base_node = t * BPT
        zv = jnp.zeros((16,), jnp.float32)

        pltpu.sync_copy(binb_hbm.at[pl.ds(0, 40)], bv.at[pl.ds(0, 40)])
        lo = bv[pl.ds(t, 16)][0]
        hi = bv[pl.ds(t + 1, 16)][0]
        lo_al = (lo // 8) * 8
        nch = (hi - lo_al + SCH2 - 1) // SCH2

        def start(c, ivb, rowsb, si, sv):
            base = lo_al + c * SCH2
            pltpu.async_copy(idx_hbm.at[pl.ds(base, SCH2)],
                             ivb.at[pl.ds(0, SCH2)], si)
            pltpu.async_copy(vals_hbm.at[pl.ds(base, SCH2)], rowsb, sv)

        def wait(c, ivb, rowsb, si, sv):
            base = lo_al + c * SCH2
            pltpu.make_async_copy(idx_hbm.at[pl.ds(base, SCH2)],
                                  ivb.at[pl.ds(0, SCH2)], si).wait()
            pltpu.make_async_copy(vals_hbm.at[pl.ds(base, SCH2)],
                                  rowsb, sv).wait()

        def process(ivb, rowsb):
            def inner(e, c2):
                r = ivb[pl.ds(e, 16)][0] - base_node
                r = jnp.where((r < 0) | (r >= BPT), BPT, r)
                for k in range(H // 16):
                    plsc.addupdate(acc.at[r, pl.ds(k * 16, 16)],
                                   rowsb[e, pl.ds(k * 16, 16)])
                return c2

            lax.fori_loop(0, SCH2, inner, 0)

        @pl.when(nch > 0)
        def _():
            start(0, iv0, rows0, si0, sv0)

        def zr(r, carry):
            for k in range(H // 16):
                acc[r, pl.ds(k * 16, 16)] = zv
            return carry

        lax.fori_loop(0, ACC_R, zr, 0)

        def body(k, carry):
            c0 = 2 * k
            c1 = 2 * k + 1

            @pl.when(c1 < nch)
            def _():
                start(c1, iv1, rows1, si1, sv1)

            wait(c0, iv0, rows0, si0, sv0)
            process(iv0, rows0)

            @pl.when(c0 + 2 < nch)
            def _():
                start(c0 + 2, iv0, rows0, si0, sv0)

            @pl.when(c1 < nch)
            def _():
                wait(c1, iv1, rows1, si1, sv1)
                process(iv1, rows1)

            return carry

        lax.fori_loop(0, (nch + 1) // 2, body, 0)
        pltpu.sync_copy(acc.at[pl.ds(0, BPT)],
                        out_hbm.at[pl.ds(base_node, BPT)])

    return gather, scatter_add


def _mm_body(x_ref, w_ref, b_ref, o_ref):
    o_ref[...] = (jnp.dot(x_ref[...], w_ref[...],
                          preferred_element_type=jnp.float32) + b_ref[...])


def _mm(x, w, b):
    n, k = x.shape
    m = w.shape[1]
    return pl.pallas_call(
        _mm_body,
        grid=(n // BN,),
        in_specs=[pl.BlockSpec((BN, k), lambda i: (i, 0)),
                  pl.BlockSpec((k, m), lambda i: (0, 0)),
                  pl.BlockSpec((1, m), lambda i: (0, 0))],
        out_specs=pl.BlockSpec((BN, m), lambda i: (i, 0)),
        out_shape=jax.ShapeDtypeStruct((n, m), jnp.float32),
    )(x, w, b.reshape(1, m))


def _mm2_body(x_ref, wa_ref, wb_ref, oa_ref, ob_ref):
    x = x_ref[...]
    oa_ref[...] = jnp.dot(x, wa_ref[...], preferred_element_type=jnp.float32)
    ob_ref[...] = jnp.dot(x, wb_ref[...], preferred_element_type=jnp.float32)


def _mm2(x, wa, wb):
    n, k = x.shape
    m = wa.shape[1]
    return pl.pallas_call(
        _mm2_body,
        grid=(n // BN,),
        in_specs=[pl.BlockSpec((BN, k), lambda i: (i, 0)),
                  pl.BlockSpec((k, m), lambda i: (0, 0)),
                  pl.BlockSpec((k, m), lambda i: (0, 0))],
        out_specs=[pl.BlockSpec((BN, m), lambda i: (i, 0)),
                   pl.BlockSpec((BN, m), lambda i: (i, 0))],
        out_shape=[jax.ShapeDtypeStruct((n, m), jnp.float32),
                   jax.ShapeDtypeStruct((n, m), jnp.float32)],
    )(x, wa, wb)


def _edge_body(xd_ref, xs_ref, ea_ref, wee_ref, w1c_ref, bee_ref, b1_ref,
               w2_ref, b2_ref, o_ref):
    wf = jnp.dot(wee_ref[...], w1c_ref[...], preferred_element_type=jnp.float32)
    bf = (jnp.dot(bee_ref[...], w1c_ref[...],
                  preferred_element_type=jnp.float32) + b1_ref[...])
    m1 = (xd_ref[...] + xs_ref[...]
          + jnp.dot(ea_ref[...], wf, preferred_element_type=jnp.float32) + bf)
    m1 = jnp.maximum(m1, 0.0)
    m2 = jnp.dot(m1, w2_ref[...], preferred_element_type=jnp.float32) + b2_ref[...]
    o_ref[...] = jnp.maximum(m2, 0.0)


def _edge_mlp(xd, xs, ea, W_ee, W1c, b_ee, b1, W2, b2):
    return pl.pallas_call(
        _edge_body,
        grid=(EP // BE,),
        in_specs=[pl.BlockSpec((BE, H), lambda i: (i, 0)),
                  pl.BlockSpec((BE, H), lambda i: (i, 0)),
                  pl.BlockSpec((BE, DE), lambda i: (i, 0)),
                  pl.BlockSpec((DE, H), lambda i: (0, 0)),
                  pl.BlockSpec((H, H), lambda i: (0, 0)),
                  pl.BlockSpec((1, H), lambda i: (0, 0)),
                  pl.BlockSpec((1, H), lambda i: (0, 0)),
                  pl.BlockSpec((H, H), lambda i: (0, 0)),
                  pl.BlockSpec((1, H), lambda i: (0, 0))],
        out_specs=pl.BlockSpec((BE, H), lambda i: (i, 0)),
        out_shape=jax.ShapeDtypeStruct((EP, H), jnp.float32),
    )(xd, xs, ea, W_ee, W1c, b_ee.reshape(1, H), b1.reshape(1, H),
      W2, b2.reshape(1, H))


def _upd_body(h_ref, a_ref, wa_ref, wb_ref, b_ref, o_ref):
    t = (jnp.dot(h_ref[...], wa_ref[...], preferred_element_type=jnp.float32)
         + jnp.dot(a_ref[...], wb_ref[...], preferred_element_type=jnp.float32)
         + b_ref[...])
    o_ref[...] = jnp.maximum(t, 0.0)


def _update(h, aggr, wa, wb, b):
    return pl.pallas_call(
        _upd_body,
        grid=(N // BN,),
        in_specs=[pl.BlockSpec((BN, H), lambda i: (i, 0)),
                  pl.BlockSpec((BN, H), lambda i: (i, 0)),
                  pl.BlockSpec((H, H), lambda i: (0, 0)),
                  pl.BlockSpec((H, H), lambda i: (0, 0)),
                  pl.BlockSpec((1, H), lambda i: (0, 0))],
        out_specs=pl.BlockSpec((BN, H), lambda i: (i, 0)),
        out_shape=jax.ShapeDtypeStruct((N, H), jnp.float32),
    )(h, aggr, wa, wb, b.reshape(1, H))


def _ro_body(h_ref, wr1_ref, br1_ref, wr2_ref, br2_ref, o_ref):
    g = jnp.sum(h_ref[...], axis=0, keepdims=True) * (1.0 / N)
    t = (jnp.dot(g, wr1_ref[...], preferred_element_type=jnp.float32)
         + br1_ref[...])
    t = jnp.maximum(t, 0.0)
    o_ref[...] = (jnp.dot(t, wr2_ref[...], preferred_element_type=jnp.float32)
                  + br2_ref[...])


def _readout(h, Wr1, br1, Wr2, br2):
    R = Wr1.shape[1]
    OUT = Wr2.shape[1]
    return pl.pallas_call(
        _ro_body,
        grid=(1,),
        in_specs=[pl.BlockSpec((N, H), lambda i: (0, 0)),
                  pl.BlockSpec((H, R), lambda i: (0, 0)),
                  pl.BlockSpec((1, R), lambda i: (0, 0)),
                  pl.BlockSpec((R, OUT), lambda i: (0, 0)),
                  pl.BlockSpec((1, OUT), lambda i: (0, 0))],
        out_specs=pl.BlockSpec((1, OUT), lambda i: (0, 0)),
        out_shape=jax.ShapeDtypeStruct((1, OUT), jnp.float32),
    )(h, Wr1, br1.reshape(1, R), Wr2, br2.reshape(1, OUT))


def kernel(x, edge_index, edge_attr, batch, W_ne, b_ne, W_ee, b_ee,
           W1_0, b1_0, W2_0, b2_0, Wu_0, bu_0,
           W1_1, b1_1, W2_1, b2_1, Wu_1, bu_1,
           W1_2, b1_2, W2_2, b2_2, Wu_2, bu_2,
           Wr1, br1, Wr2, br2):
    src = edge_index[0]
    dst = edge_index[1]
    perm = jnp.argsort(dst).astype(jnp.int32)
    dst_srt = jnp.take(dst, perm)
    src_srt = jnp.take(src, perm)
    binb = jnp.searchsorted(
        dst_srt, jnp.arange(33, dtype=jnp.int32) * BPT).astype(jnp.int32)
    binb = jnp.concatenate([binb, jnp.full((7,), E, jnp.int32)])

    pad = EP - E
    zpad = jnp.zeros((pad,), jnp.int32)
    dst_g = jnp.concatenate([dst_srt, zpad])
    src_g = jnp.concatenate([src_srt, zpad])
    dst_sc = jnp.concatenate([dst_srt, jnp.full((pad,), 1 << 30, jnp.int32)])
    ea_srt = jnp.concatenate(
        [jnp.take(edge_attr, perm, axis=0), jnp.zeros((pad, DE), jnp.float32)])

    _gather_nodes, _sc_scatter_add = _sc_kernels()
    h = _mm(x, W_ne, b_ne)
    layers = [(W1_0, b1_0, W2_0, b2_0, Wu_0, bu_0),
              (W1_1, b1_1, W2_1, b2_1, Wu_1, bu_1),
              (W1_2, b1_2, W2_2, b2_2, Wu_2, bu_2)]
    for (W1, b1, W2, b2, Wu, bu) in layers:
        Hd, Hs = _mm2(h, W1[:H], W1[H:2 * H])
        Xd = _gather_nodes(Hd, dst_g)
        Xs = _gather_nodes(Hs, src_g)
        m2 = _edge_mlp(Xd, Xs, ea_srt, W_ee, W1[2 * H:], b_ee, b1, W2, b2)
        aggr = _sc_scatter_add(m2, dst_sc, binb)
        h = _update(h, aggr[:N], Wu[:H], Wu[H:], bu)
    return _readout(h, Wr1, br1, Wr2, br2)


# speedup vs baseline: 1.4553x; 1.2345x over previous
import functools

import jax
import jax.numpy as jnp
from jax import lax
from jax.experimental import pallas as pl
from jax.experimental.pallas import tpu as pltpu
from jax.experimental.pallas import tpu_sc as plsc

N = 10000
E = 160000
DE = 16
H = 256

NC, NS = 2, 16
NW = NC * NS
CH = 128
EP = 163840
EPW = EP // NW
GCH = EPW // CH

BPT = 320
ACC_R = BPT + 8
SCH2 = 64
NPAD = NW * BPT

BN = 2000
BE = 1280


@functools.cache
def _sc_kernels():
    mesh = plsc.VectorSubcoreMesh(core_axis_name="c", subcore_axis_name="s")

    CH3 = 64
    NCHW = EPW // CH3
    HB = NCHW // 2

    @functools.partial(
        pl.kernel,
        mesh=mesh,
        out_type=jax.ShapeDtypeStruct((EP, H), jnp.float32),
        scratch_types=[
            pltpu.VMEM((CH3,), jnp.int32),
            pltpu.VMEM((CH3,), jnp.int32),
            pltpu.VMEM((CH3,), jnp.int32),
            pltpu.VMEM((CH3,), jnp.int32),
            pltpu.VMEM((CH3, H), jnp.float32),
            pltpu.VMEM((CH3, H), jnp.float32),
            pltpu.VMEM((CH3, H), jnp.float32),
            pltpu.VMEM((CH3, H), jnp.float32),
            pltpu.VMEM((CH3, H), jnp.float32),
            pltpu.VMEM((CH3, H), jnp.float32),
            pltpu.SemaphoreType.DMA,
            pltpu.SemaphoreType.DMA,
            pltpu.SemaphoreType.DMA,
            pltpu.SemaphoreType.DMA,
            pltpu.SemaphoreType.DMA,
            pltpu.SemaphoreType.DMA,
        ],
    )
    def gather_sum(td_hbm, ts_hbm, idxd_hbm, idxs_hbm, out_hbm,
                   id0, id1, is0, is1, rd0, rd1, rs0, rs1, ou0, ou1,
                   sd0, sd1, ss0, ss1, so0, so1):
        wid = lax.axis_index("s") * NC + lax.axis_index("c")
        w0 = wid * EPW

        def start_g(c, idb, isb, rdb, rsb, sdb, ssb):
            base = w0 + c * CH3
            pltpu.sync_copy(idxd_hbm.at[pl.ds(base, CH3)], idb)
            pltpu.sync_copy(idxs_hbm.at[pl.ds(base, CH3)], isb)
            pltpu.async_copy(td_hbm.at[idb], rdb, sdb)
            pltpu.async_copy(ts_hbm.at[isb], rsb, ssb)

        def wait_g(idb, isb, rdb, rsb, sdb, ssb):
            pltpu.make_async_copy(td_hbm.at[idb], rdb, sdb).wait()
            pltpu.make_async_copy(ts_hbm.at[isb], rsb, ssb).wait()

        def sum_rows(rdb, rsb, oub):
            def ebody(e, carry):
                for k in range(H // 16):
                    oub[e, pl.ds(k * 16, 16)] = (
                        rdb[e, pl.ds(k * 16, 16)] + rsb[e, pl.ds(k * 16, 16)])
                return carry

            lax.fori_loop(0, CH3, ebody, 0)

        def start_o(c, oub, sob):
            pltpu.async_copy(oub, out_hbm.at[pl.ds(w0 + c * CH3, CH3)], sob)

        def wait_o(c, oub, sob):
            pltpu.make_async_copy(
                oub, out_hbm.at[pl.ds(w0 + c * CH3, CH3)], sob).wait()

        start_g(0, id0, is0, rd0, rs0, sd0, ss0)
        start_g(1, id1, is1, rd1, rs1, sd1, ss1)

        def body(k, carry):
            c0 = 2 * k
            c1 = 2 * k + 1
            wait_g(id0, is0, rd0, rs0, sd0, ss0)

            @pl.when(k > 0)
            def _():
                wait_o(c0 - 2, ou0, so0)

            sum_rows(rd0, rs0, ou0)
            start_o(c0, ou0, so0)

            @pl.when(c0 + 2 < NCHW)
            def _():
                start_g(c0 + 2, id0, is0, rd0, rs0, sd0, ss0)

            wait_g(id1, is1, rd1, rs1, sd1, ss1)

            @pl.when(k > 0)
            def _():
                wait_o(c1 - 2, ou1, so1)

            sum_rows(rd1, rs1, ou1)
            start_o(c1, ou1, so1)

            @pl.when(c1 + 2 < NCHW)
            def _():
                start_g(c1 + 2, id1, is1, rd1, rs1, sd1, ss1)

            return carry

        lax.fori_loop(0, HB, body, 0)
        wait_o(NCHW - 2, ou0, so0)
        wait_o(NCHW - 1, ou1, so1)

    @functools.partial(
        pl.kernel,
        mesh=mesh,
        out_type=jax.ShapeDtypeStruct((NPAD, H), jnp.float32),
        scratch_types=[
            pltpu.VMEM((56,), jnp.int32),
            pltpu.VMEM((SCH2 + 16,), jnp.int32),
            pltpu.VMEM((SCH2 + 16,), jnp.int32),
            pltpu.VMEM((SCH2, H), jnp.float32),
            pltpu.VMEM((SCH2, H), jnp.float32),
            pltpu.VMEM((ACC_R, H), jnp.float32),
            pltpu.SemaphoreType.DMA,
            pltpu.SemaphoreType.DMA,
            pltpu.SemaphoreType.DMA,
            pltpu.SemaphoreType.DMA,
        ],
    )
    def scatter_add(vals_hbm, idx_hbm, binb_hbm, out_hbm, bv, iv0, iv1,
                    rows0, rows1, acc, si0, si1, sv0, sv1):
        t = lax.axis_index("s") * NC + lax.axis_index("c")
        base_node = t * BPT
        zv = jnp.zeros((16,), jnp.float32)

        pltpu.sync_copy(binb_hbm.at[pl.ds(0, 40)], bv.at[pl.ds(0, 40)])
        lo = bv[pl.ds(t, 16)][0]
        hi = bv[pl.ds(t + 1, 16)][0]
        lo_al = (lo // 8) * 8
        nch = (hi - lo_al + SCH2 - 1) // SCH2

        def start(c, ivb, rowsb, si, sv):
            base = lo_al + c * SCH2
            pltpu.async_copy(idx_hbm.at[pl.ds(base, SCH2)],
                             ivb.at[pl.ds(0, SCH2)], si)
            pltpu.async_copy(vals_hbm.at[pl.ds(base, SCH2)], rowsb, sv)

        def wait(c, ivb, rowsb, si, sv):
            base = lo_al + c * SCH2
            pltpu.make_async_copy(idx_hbm.at[pl.ds(base, SCH2)],
                                  ivb.at[pl.ds(0, SCH2)], si).wait()
            pltpu.make_async_copy(vals_hbm.at[pl.ds(base, SCH2)],
                                  rowsb, sv).wait()

        def process(ivb, rowsb):
            def inner(e, c2):
                r = ivb[pl.ds(e, 16)][0] - base_node
                r = jnp.where((r < 0) | (r >= BPT), BPT, r)
                for k in range(H // 16):
                    plsc.addupdate(acc.at[r, pl.ds(k * 16, 16)],
                                   rowsb[e, pl.ds(k * 16, 16)])
                return c2

            lax.fori_loop(0, SCH2, inner, 0)

        @pl.when(nch > 0)
        def _():
            start(0, iv0, rows0, si0, sv0)

        def zr(r, carry):
            for k in range(H // 16):
                acc[r, pl.ds(k * 16, 16)] = zv
            return carry

        lax.fori_loop(0, ACC_R, zr, 0)

        def body(k, carry):
            c0 = 2 * k
            c1 = 2 * k + 1

            @pl.when(c1 < nch)
            def _():
                start(c1, iv1, rows1, si1, sv1)

            wait(c0, iv0, rows0, si0, sv0)
            process(iv0, rows0)

            @pl.when(c0 + 2 < nch)
            def _():
                start(c0 + 2, iv0, rows0, si0, sv0)

            @pl.when(c1 < nch)
            def _():
                wait(c1, iv1, rows1, si1, sv1)
                process(iv1, rows1)

            return carry

        lax.fori_loop(0, (nch + 1) // 2, body, 0)
        pltpu.sync_copy(acc.at[pl.ds(0, BPT)],
                        out_hbm.at[pl.ds(base_node, BPT)])

    return gather_sum, scatter_add


def _mm_body(x_ref, w_ref, b_ref, o_ref):
    o_ref[...] = (jnp.dot(x_ref[...], w_ref[...],
                          preferred_element_type=jnp.float32) + b_ref[...])


def _mm(x, w, b):
    n, k = x.shape
    m = w.shape[1]
    return pl.pallas_call(
        _mm_body,
        grid=(n // BN,),
        in_specs=[pl.BlockSpec((BN, k), lambda i: (i, 0)),
                  pl.BlockSpec((k, m), lambda i: (0, 0)),
                  pl.BlockSpec((1, m), lambda i: (0, 0))],
        out_specs=pl.BlockSpec((BN, m), lambda i: (i, 0)),
        out_shape=jax.ShapeDtypeStruct((n, m), jnp.float32),
    )(x, w, b.reshape(1, m))


def _mm2_body(x_ref, wa_ref, wb_ref, oa_ref, ob_ref):
    x = x_ref[...]
    oa_ref[...] = jnp.dot(x, wa_ref[...], preferred_element_type=jnp.float32)
    ob_ref[...] = jnp.dot(x, wb_ref[...], preferred_element_type=jnp.float32)


def _mm2(x, wa, wb):
    n, k = x.shape
    m = wa.shape[1]
    return pl.pallas_call(
        _mm2_body,
        grid=(n // BN,),
        in_specs=[pl.BlockSpec((BN, k), lambda i: (i, 0)),
                  pl.BlockSpec((k, m), lambda i: (0, 0)),
                  pl.BlockSpec((k, m), lambda i: (0, 0))],
        out_specs=[pl.BlockSpec((BN, m), lambda i: (i, 0)),
                   pl.BlockSpec((BN, m), lambda i: (i, 0))],
        out_shape=[jax.ShapeDtypeStruct((n, m), jnp.float32),
                   jax.ShapeDtypeStruct((n, m), jnp.float32)],
    )(x, wa, wb)


def _edge_body(g_ref, ea_ref, wee_ref, w1c_ref, bee_ref, b1_ref,
               w2_ref, b2_ref, o_ref):
    wf = jnp.dot(wee_ref[...], w1c_ref[...], preferred_element_type=jnp.float32)
    bf = (jnp.dot(bee_ref[...], w1c_ref[...],
                  preferred_element_type=jnp.float32) + b1_ref[...])
    m1 = (g_ref[...]
          + jnp.dot(ea_ref[...], wf, preferred_element_type=jnp.float32) + bf)
    m1 = jnp.maximum(m1, 0.0)
    m2 = jnp.dot(m1, w2_ref[...], preferred_element_type=jnp.float32) + b2_ref[...]
    o_ref[...] = jnp.maximum(m2, 0.0)


def _edge_mlp(g, ea, W_ee, W1c, b_ee, b1, W2, b2):
    return pl.pallas_call(
        _edge_body,
        grid=(EP // BE,),
        in_specs=[pl.BlockSpec((BE, H), lambda i: (i, 0)),
                  pl.BlockSpec((BE, DE), lambda i: (i, 0)),
                  pl.BlockSpec((DE, H), lambda i: (0, 0)),
                  pl.BlockSpec((H, H), lambda i: (0, 0)),
                  pl.BlockSpec((1, H), lambda i: (0, 0)),
                  pl.BlockSpec((1, H), lambda i: (0, 0)),
                  pl.BlockSpec((H, H), lambda i: (0, 0)),
                  pl.BlockSpec((1, H), lambda i: (0, 0))],
        out_specs=pl.BlockSpec((BE, H), lambda i: (i, 0)),
        out_shape=jax.ShapeDtypeStruct((EP, H), jnp.float32),
    )(g, ea, W_ee, W1c, b_ee.reshape(1, H), b1.reshape(1, H),
      W2, b2.reshape(1, H))


def _upd_body(h_ref, a_ref, wa_ref, wb_ref, b_ref, o_ref):
    t = (jnp.dot(h_ref[...], wa_ref[...], preferred_element_type=jnp.float32)
         + jnp.dot(a_ref[...], wb_ref[...], preferred_element_type=jnp.float32)
         + b_ref[...])
    o_ref[...] = jnp.maximum(t, 0.0)


def _update(h, aggr, wa, wb, b):
    return pl.pallas_call(
        _upd_body,
        grid=(N // BN,),
        in_specs=[pl.BlockSpec((BN, H), lambda i: (i, 0)),
                  pl.BlockSpec((BN, H), lambda i: (i, 0)),
                  pl.BlockSpec((H, H), lambda i: (0, 0)),
                  pl.BlockSpec((H, H), lambda i: (0, 0)),
                  pl.BlockSpec((1, H), lambda i: (0, 0))],
        out_specs=pl.BlockSpec((BN, H), lambda i: (i, 0)),
        out_shape=jax.ShapeDtypeStruct((N, H), jnp.float32),
    )(h, aggr, wa, wb, b.reshape(1, H))


def _ro_body(h_ref, wr1_ref, br1_ref, wr2_ref, br2_ref, o_ref):
    g = jnp.sum(h_ref[...], axis=0, keepdims=True) * (1.0 / N)
    t = (jnp.dot(g, wr1_ref[...], preferred_element_type=jnp.float32)
         + br1_ref[...])
    t = jnp.maximum(t, 0.0)
    o_ref[...] = (jnp.dot(t, wr2_ref[...], preferred_element_type=jnp.float32)
                  + br2_ref[...])


def _readout(h, Wr1, br1, Wr2, br2):
    R = Wr1.shape[1]
    OUT = Wr2.shape[1]
    return pl.pallas_call(
        _ro_body,
        grid=(1,),
        in_specs=[pl.BlockSpec((N, H), lambda i: (0, 0)),
                  pl.BlockSpec((H, R), lambda i: (0, 0)),
                  pl.BlockSpec((1, R), lambda i: (0, 0)),
                  pl.BlockSpec((R, OUT), lambda i: (0, 0)),
                  pl.BlockSpec((1, OUT), lambda i: (0, 0))],
        out_specs=pl.BlockSpec((1, OUT), lambda i: (0, 0)),
        out_shape=jax.ShapeDtypeStruct((1, OUT), jnp.float32),
    )(h, Wr1, br1.reshape(1, R), Wr2, br2.reshape(1, OUT))


def kernel(x, edge_index, edge_attr, batch, W_ne, b_ne, W_ee, b_ee,
           W1_0, b1_0, W2_0, b2_0, Wu_0, bu_0,
           W1_1, b1_1, W2_1, b2_1, Wu_1, bu_1,
           W1_2, b1_2, W2_2, b2_2, Wu_2, bu_2,
           Wr1, br1, Wr2, br2):
    src = edge_index[0]
    dst = edge_index[1]
    perm = jnp.argsort(dst).astype(jnp.int32)
    dst_srt = jnp.take(dst, perm)
    src_srt = jnp.take(src, perm)
    binb = jnp.searchsorted(
        dst_srt, jnp.arange(33, dtype=jnp.int32) * BPT).astype(jnp.int32)
    binb = jnp.concatenate([binb, jnp.full((7,), E, jnp.int32)])

    pad = EP - E
    zpad = jnp.zeros((pad,), jnp.int32)
    dst_g = jnp.concatenate([dst_srt, zpad])
    src_g = jnp.concatenate([src_srt, zpad])
    dst_sc = jnp.concatenate([dst_srt, jnp.full((pad,), 1 << 30, jnp.int32)])
    ea_srt = jnp.concatenate(
        [jnp.take(edge_attr, perm, axis=0), jnp.zeros((pad, DE), jnp.float32)])

    _gather_sum, _sc_scatter_add = _sc_kernels()
    h = _mm(x, W_ne, b_ne)
    layers = [(W1_0, b1_0, W2_0, b2_0, Wu_0, bu_0),
              (W1_1, b1_1, W2_1, b2_1, Wu_1, bu_1),
              (W1_2, b1_2, W2_2, b2_2, Wu_2, bu_2)]
    for (W1, b1, W2, b2, Wu, bu) in layers:
        Hd, Hs = _mm2(h, W1[:H], W1[H:2 * H])
        Gsum = _gather_sum(Hd, Hs, dst_g, src_g)
        m2 = _edge_mlp(Gsum, ea_srt, W_ee, W1[2 * H:], b_ee, b1, W2, b2)
        aggr = _sc_scatter_add(m2, dst_sc, binb)
        h = _update(h, aggr[:N], Wu[:H], Wu[H:], bu)
    return _readout(h, Wr1, br1, Wr2, br2)


# speedup vs baseline: 1.4647x; 1.0065x over previous
import functools

import jax
import jax.numpy as jnp
from jax import lax
from jax.experimental import pallas as pl
from jax.experimental.pallas import tpu as pltpu
from jax.experimental.pallas import tpu_sc as plsc

N = 10000
E = 160000
DE = 16
H = 256

NC, NS = 2, 16
NW = NC * NS
CH = 128
EP = 163840
EPW = EP // NW
GCH = EPW // CH

BPT = 320
ACC_R = BPT + 8
SCH2 = 64
NPAD = NW * BPT

BN = 2000
BE = 1280


@functools.cache
def _sc_kernels():
    mesh = plsc.VectorSubcoreMesh(core_axis_name="c", subcore_axis_name="s")

    CH3 = 64
    NCHW = EPW // CH3
    HB = NCHW // 2

    @functools.partial(
        pl.kernel,
        mesh=mesh,
        out_type=jax.ShapeDtypeStruct((EP, H), jnp.float32),
        scratch_types=[
            pltpu.VMEM((CH3,), jnp.int32),
            pltpu.VMEM((CH3,), jnp.int32),
            pltpu.VMEM((CH3,), jnp.int32),
            pltpu.VMEM((CH3,), jnp.int32),
            pltpu.VMEM((CH3, H), jnp.float32),
            pltpu.VMEM((CH3, H), jnp.float32),
            pltpu.VMEM((CH3, H), jnp.float32),
            pltpu.VMEM((CH3, H), jnp.float32),
            pltpu.VMEM((CH3, H), jnp.float32),
            pltpu.VMEM((CH3, H), jnp.float32),
            pltpu.SemaphoreType.DMA,
            pltpu.SemaphoreType.DMA,
            pltpu.SemaphoreType.DMA,
            pltpu.SemaphoreType.DMA,
            pltpu.SemaphoreType.DMA,
            pltpu.SemaphoreType.DMA,
        ],
    )
    def gather_sum(td_hbm, ts_hbm, idxd_hbm, idxs_hbm, out_hbm,
                   id0, id1, is0, is1, rd0, rd1, rs0, rs1, ou0, ou1,
                   sd0, sd1, ss0, ss1, so0, so1):
        wid = lax.axis_index("s") * NC + lax.axis_index("c")
        w0 = wid * EPW

        def start_g(c, idb, isb, rdb, rsb, sdb, ssb):
            base = w0 + c * CH3
            pltpu.sync_copy(idxd_hbm.at[pl.ds(base, CH3)], idb)
            pltpu.sync_copy(idxs_hbm.at[pl.ds(base, CH3)], isb)
            pltpu.async_copy(td_hbm.at[idb], rdb, sdb)
            pltpu.async_copy(ts_hbm.at[isb], rsb, ssb)

        def wait_g(idb, isb, rdb, rsb, sdb, ssb):
            pltpu.make_async_copy(td_hbm.at[idb], rdb, sdb).wait()
            pltpu.make_async_copy(ts_hbm.at[isb], rsb, ssb).wait()

        def sum_rows(rdb, rsb, oub):
            def ebody(e, carry):
                for k in range(H // 16):
                    oub[e, pl.ds(k * 16, 16)] = (
                        rdb[e, pl.ds(k * 16, 16)] + rsb[e, pl.ds(k * 16, 16)])
                return carry

            lax.fori_loop(0, CH3, ebody, 0)

        def start_o(c, oub, sob):
            pltpu.async_copy(oub, out_hbm.at[pl.ds(w0 + c * CH3, CH3)], sob)

        def wait_o(c, oub, sob):
            pltpu.make_async_copy(
                oub, out_hbm.at[pl.ds(w0 + c * CH3, CH3)], sob).wait()

        start_g(0, id0, is0, rd0, rs0, sd0, ss0)
        start_g(1, id1, is1, rd1, rs1, sd1, ss1)

        def body(k, carry):
            c0 = 2 * k
            c1 = 2 * k + 1
            wait_g(id0, is0, rd0, rs0, sd0, ss0)

            @pl.when(k > 0)
            def _():
                wait_o(c0 - 2, ou0, so0)

            sum_rows(rd0, rs0, ou0)
            start_o(c0, ou0, so0)

            @pl.when(c0 + 2 < NCHW)
            def _():
                start_g(c0 + 2, id0, is0, rd0, rs0, sd0, ss0)

            wait_g(id1, is1, rd1, rs1, sd1, ss1)

            @pl.when(k > 0)
            def _():
                wait_o(c1 - 2, ou1, so1)

            sum_rows(rd1, rs1, ou1)
            start_o(c1, ou1, so1)

            @pl.when(c1 + 2 < NCHW)
            def _():
                start_g(c1 + 2, id1, is1, rd1, rs1, sd1, ss1)

            return carry

        lax.fori_loop(0, HB, body, 0)
        wait_o(NCHW - 2, ou0, so0)
        wait_o(NCHW - 1, ou1, so1)

    @functools.partial(
        pl.kernel,
        mesh=mesh,
        out_type=jax.ShapeDtypeStruct((NPAD, H), jnp.float32),
        scratch_types=[
            pltpu.VMEM((56,), jnp.int32),
            pltpu.VMEM((SCH2 + 16,), jnp.int32),
            pltpu.VMEM((SCH2 + 16,), jnp.int32),
            pltpu.VMEM((SCH2, H), jnp.float32),
            pltpu.VMEM((SCH2, H), jnp.float32),
            pltpu.VMEM((ACC_R, H), jnp.float32),
            pltpu.SemaphoreType.DMA,
            pltpu.SemaphoreType.DMA,
            pltpu.SemaphoreType.DMA,
            pltpu.SemaphoreType.DMA,
        ],
    )
    def scatter_add(vals_hbm, idx_hbm, binb_hbm, out_hbm, bv, iv0, iv1,
                    rows0, rows1, acc, si0, si1, sv0, sv1):
        t = lax.axis_index("s") * NC + lax.axis_index("c")
        base_node = t * BPT
        zv = jnp.zeros((16,), jnp.float32)

        pltpu.sync_copy(binb_hbm.at[pl.ds(0, 40)], bv.at[pl.ds(0, 40)])
        lo = bv[pl.ds(t, 16)][0]
        hi = bv[pl.ds(t + 1, 16)][0]
        lo_al = (lo // 8) * 8
        nch = (hi - lo_al + SCH2 - 1) // SCH2

        def start(c, ivb, rowsb, si, sv):
            base = lo_al + c * SCH2
            pltpu.async_copy(idx_hbm.at[pl.ds(base, SCH2)],
                             ivb.at[pl.ds(0, SCH2)], si)
            pltpu.async_copy(vals_hbm.at[pl.ds(base, SCH2)], rowsb, sv)

        def wait(c, ivb, rowsb, si, sv):
            base = lo_al + c * SCH2
            pltpu.make_async_copy(idx_hbm.at[pl.ds(base, SCH2)],
                                  ivb.at[pl.ds(0, SCH2)], si).wait()
            pltpu.make_async_copy(vals_hbm.at[pl.ds(base, SCH2)],
                                  rowsb, sv).wait()

        def process(ivb, rowsb):
            for j in range(SCH2 // 16):
                v = ivb[pl.ds(j * 16, 16)] - base_node
                oob = (v < 0) | (v >= BPT)
                ivb[pl.ds(j * 16, 16)] = jnp.where(oob, BPT, v)

            def inner(q, c2):
                for u in range(4):
                    e = q * 4 + u
                    r = ivb[pl.ds(e, 16)][0]
                    for k in range(H // 16):
                        plsc.addupdate(acc.at[r, pl.ds(k * 16, 16)],
                                       rowsb[e, pl.ds(k * 16, 16)])
                return c2

            lax.fori_loop(0, SCH2 // 4, inner, 0)

        @pl.when(nch > 0)
        def _():
            start(0, iv0, rows0, si0, sv0)

        def zr(r, carry):
            for k in range(H // 16):
                acc[r, pl.ds(k * 16, 16)] = zv
            return carry

        lax.fori_loop(0, ACC_R, zr, 0)

        def body(k, carry):
            c0 = 2 * k
            c1 = 2 * k + 1

            @pl.when(c1 < nch)
            def _():
                start(c1, iv1, rows1, si1, sv1)

            wait(c0, iv0, rows0, si0, sv0)
            process(iv0, rows0)

            @pl.when(c0 + 2 < nch)
            def _():
                start(c0 + 2, iv0, rows0, si0, sv0)

            @pl.when(c1 < nch)
            def _():
                wait(c1, iv1, rows1, si1, sv1)
                process(iv1, rows1)

            return carry

        lax.fori_loop(0, (nch + 1) // 2, body, 0)
        pltpu.sync_copy(acc.at[pl.ds(0, BPT)],
                        out_hbm.at[pl.ds(base_node, BPT)])

    return gather_sum, scatter_add


def _mm_body(x_ref, w_ref, b_ref, o_ref):
    o_ref[...] = (jnp.dot(x_ref[...], w_ref[...],
                          preferred_element_type=jnp.float32) + b_ref[...])


def _mm(x, w, b):
    n, k = x.shape
    m = w.shape[1]
    return pl.pallas_call(
        _mm_body,
        grid=(n // BN,),
        in_specs=[pl.BlockSpec((BN, k), lambda i: (i, 0)),
                  pl.BlockSpec((k, m), lambda i: (0, 0)),
                  pl.BlockSpec((1, m), lambda i: (0, 0))],
        out_specs=pl.BlockSpec((BN, m), lambda i: (i, 0)),
        out_shape=jax.ShapeDtypeStruct((n, m), jnp.float32),
    )(x, w, b.reshape(1, m))


def _mm2_body(x_ref, wa_ref, wb_ref, oa_ref, ob_ref):
    x = x_ref[...]
    oa_ref[...] = jnp.dot(x, wa_ref[...], preferred_element_type=jnp.float32)
    ob_ref[...] = jnp.dot(x, wb_ref[...], preferred_element_type=jnp.float32)


def _mm2(x, wa, wb):
    n, k = x.shape
    m = wa.shape[1]
    return pl.pallas_call(
        _mm2_body,
        grid=(n // BN,),
        in_specs=[pl.BlockSpec((BN, k), lambda i: (i, 0)),
                  pl.BlockSpec((k, m), lambda i: (0, 0)),
                  pl.BlockSpec((k, m), lambda i: (0, 0))],
        out_specs=[pl.BlockSpec((BN, m), lambda i: (i, 0)),
                   pl.BlockSpec((BN, m), lambda i: (i, 0))],
        out_shape=[jax.ShapeDtypeStruct((n, m), jnp.float32),
                   jax.ShapeDtypeStruct((n, m), jnp.float32)],
    )(x, wa, wb)


def _edge_body(g_ref, ea_ref, wee_ref, w1c_ref, bee_ref, b1_ref,
               w2_ref, b2_ref, o_ref):
    wf = jnp.dot(wee_ref[...], w1c_ref[...], preferred_element_type=jnp.float32)
    bf = (jnp.dot(bee_ref[...], w1c_ref[...],
                  preferred_element_type=jnp.float32) + b1_ref[...])
    m1 = (g_ref[...]
          + jnp.dot(ea_ref[...], wf, preferred_element_type=jnp.float32) + bf)
    m1 = jnp.maximum(m1, 0.0)
    m2 = jnp.dot(m1, w2_ref[...], preferred_element_type=jnp.float32) + b2_ref[...]
    o_ref[...] = jnp.maximum(m2, 0.0)


def _edge_mlp(g, ea, W_ee, W1c, b_ee, b1, W2, b2):
    return pl.pallas_call(
        _edge_body,
        grid=(EP // BE,),
        in_specs=[pl.BlockSpec((BE, H), lambda i: (i, 0)),
                  pl.BlockSpec((BE, DE), lambda i: (i, 0)),
                  pl.BlockSpec((DE, H), lambda i: (0, 0)),
                  pl.BlockSpec((H, H), lambda i: (0, 0)),
                  pl.BlockSpec((1, H), lambda i: (0, 0)),
                  pl.BlockSpec((1, H), lambda i: (0, 0)),
                  pl.BlockSpec((H, H), lambda i: (0, 0)),
                  pl.BlockSpec((1, H), lambda i: (0, 0))],
        out_specs=pl.BlockSpec((BE, H), lambda i: (i, 0)),
        out_shape=jax.ShapeDtypeStruct((EP, H), jnp.float32),
    )(g, ea, W_ee, W1c, b_ee.reshape(1, H), b1.reshape(1, H),
      W2, b2.reshape(1, H))


def _upd_body(h_ref, a_ref, wa_ref, wb_ref, b_ref, o_ref):
    t = (jnp.dot(h_ref[...], wa_ref[...], preferred_element_type=jnp.float32)
         + jnp.dot(a_ref[...], wb_ref[...], preferred_element_type=jnp.float32)
         + b_ref[...])
    o_ref[...] = jnp.maximum(t, 0.0)


def _update(h, aggr, wa, wb, b):
    return pl.pallas_call(
        _upd_body,
        grid=(N // BN,),
        in_specs=[pl.BlockSpec((BN, H), lambda i: (i, 0)),
                  pl.BlockSpec((BN, H), lambda i: (i, 0)),
                  pl.BlockSpec((H, H), lambda i: (0, 0)),
                  pl.BlockSpec((H, H), lambda i: (0, 0)),
                  pl.BlockSpec((1, H), lambda i: (0, 0))],
        out_specs=pl.BlockSpec((BN, H), lambda i: (i, 0)),
        out_shape=jax.ShapeDtypeStruct((N, H), jnp.float32),
    )(h, aggr, wa, wb, b.reshape(1, H))


def _ro_body(h_ref, wr1_ref, br1_ref, wr2_ref, br2_ref, o_ref):
    g = jnp.sum(h_ref[...], axis=0, keepdims=True) * (1.0 / N)
    t = (jnp.dot(g, wr1_ref[...], preferred_element_type=jnp.float32)
         + br1_ref[...])
    t = jnp.maximum(t, 0.0)
    o_ref[...] = (jnp.dot(t, wr2_ref[...], preferred_element_type=jnp.float32)
                  + br2_ref[...])


def _readout(h, Wr1, br1, Wr2, br2):
    R = Wr1.shape[1]
    OUT = Wr2.shape[1]
    return pl.pallas_call(
        _ro_body,
        grid=(1,),
        in_specs=[pl.BlockSpec((N, H), lambda i: (0, 0)),
                  pl.BlockSpec((H, R), lambda i: (0, 0)),
                  pl.BlockSpec((1, R), lambda i: (0, 0)),
                  pl.BlockSpec((R, OUT), lambda i: (0, 0)),
                  pl.BlockSpec((1, OUT), lambda i: (0, 0))],
        out_specs=pl.BlockSpec((1, OUT), lambda i: (0, 0)),
        out_shape=jax.ShapeDtypeStruct((1, OUT), jnp.float32),
    )(h, Wr1, br1.reshape(1, R), Wr2, br2.reshape(1, OUT))


def kernel(x, edge_index, edge_attr, batch, W_ne, b_ne, W_ee, b_ee,
           W1_0, b1_0, W2_0, b2_0, Wu_0, bu_0,
           W1_1, b1_1, W2_1, b2_1, Wu_1, bu_1,
           W1_2, b1_2, W2_2, b2_2, Wu_2, bu_2,
           Wr1, br1, Wr2, br2):
    src = edge_index[0]
    dst = edge_index[1]
    perm = jnp.argsort(dst).astype(jnp.int32)
    dst_srt = jnp.take(dst, perm)
    src_srt = jnp.take(src, perm)
    binb = jnp.searchsorted(
        dst_srt, jnp.arange(33, dtype=jnp.int32) * BPT).astype(jnp.int32)
    binb = jnp.concatenate([binb, jnp.full((7,), E, jnp.int32)])

    pad = EP - E
    zpad = jnp.zeros((pad,), jnp.int32)
    dst_g = jnp.concatenate([dst_srt, zpad])
    src_g = jnp.concatenate([src_srt, zpad])
    dst_sc = jnp.concatenate([dst_srt, jnp.full((pad,), 1 << 30, jnp.int32)])
    ea_srt = jnp.concatenate(
        [jnp.take(edge_attr, perm, axis=0), jnp.zeros((pad, DE), jnp.float32)])

    _gather_sum, _sc_scatter_add = _sc_kernels()
    h = _mm(x, W_ne, b_ne)
    layers = [(W1_0, b1_0, W2_0, b2_0, Wu_0, bu_0),
              (W1_1, b1_1, W2_1, b2_1, Wu_1, bu_1),
              (W1_2, b1_2, W2_2, b2_2, Wu_2, bu_2)]
    for (W1, b1, W2, b2, Wu, bu) in layers:
        Hd, Hs = _mm2(h, W1[:H], W1[H:2 * H])
        Gsum = _gather_sum(Hd, Hs, dst_g, src_g)
        m2 = _edge_mlp(Gsum, ea_srt, W_ee, W1[2 * H:], b_ee, b1, W2, b2)
        aggr = _sc_scatter_add(m2, dst_sc, binb)
        h = _update(h, aggr[:N], Wu[:H], Wu[H:], bu)
    return _readout(h, Wr1, br1, Wr2, br2)
